# Initial kernel scaffold; baseline (speedup 1.0000x reference)
import jax
import jax.numpy as jnp
from jax.experimental import pallas as pl


def kernel(aspect, center, pairs, negs, offsets, lists):
    raise NotImplementedError("write your pallas kernel here")


# speedup vs baseline: 2.3880x; 2.3880x over previous
import functools

import jax
import jax.numpy as jnp
import numpy as np
from jax.experimental import pallas as pl
from jax.experimental.pallas import tpu as pltpu


def _log_sig(x):
    return jnp.minimum(x, 0.0) - jnp.log(1.0 + jnp.exp(-jnp.abs(x)))


def _chunk_sum_mat(d, chunks):
    m = np.zeros((chunks * d, chunks), np.float32)
    for k in range(chunks):
        m[k * d:(k + 1) * d, k] = 1.0
    return m


def _loss_kernel(ctr_ref, bagg_ref, ctx_ref, neg_ref, sum_a_ref, out_ref, *,
                 num_aspects, dim, num_negs, bag, inv_total):
    A, D, NN = num_aspects, dim, num_negs
    f32 = jnp.float32

    bg = bagg_ref[...]
    pooled = bg[0]
    for j in range(1, bag):
        pooled = pooled + bg[j]
    pooled = pooled * (1.0 / bag)

    ctr = ctr_ref[...]
    ct = jnp.concatenate([ctr] * A, axis=-1)
    ctx = ctx_ref[...]
    neg = neg_ref[...]
    TB = ctr.shape[0]

    slabs = [pooled * ct, ctx * ct]
    for n in range(NN):
        slabs.append(neg[:, n * A * D:(n + 1) * A * D] * ct)
    red = jnp.dot(jnp.concatenate(slabs, axis=0), sum_a_ref[...],
                  preferred_element_type=f32)

    asp_score = red[:TB]
    sp = red[TB:2 * TB]
    score_pos = -_log_sig(sp)
    score_neg = jnp.zeros_like(sp)
    for n in range(NN):
        score_neg = score_neg - _log_sig(-red[(2 + n) * TB:(3 + n) * TB])

    m = jnp.max(asp_score, axis=-1, keepdims=True)
    e = jnp.exp(asp_score - m)
    w = e / jnp.sum(e, axis=-1, keepdims=True)

    tile_sum = jnp.sum(w * (score_pos + score_neg)) * inv_total
    out_ref[...] = jnp.full(out_ref.shape, tile_sum, f32)


def _reg_kernel(emb_ref, out_ref, *, num_aspects, threshold, eps):
    A = num_aspects
    x = emb_ref[...]
    norms = []
    for a in range(A):
        norms.append(jnp.sqrt(jnp.sum(x[a] * x[a], axis=-1, keepdims=True)))
    acc = jnp.zeros((), jnp.float32)
    for a in range(A):
        for b in range(a + 1, A):
            d = jnp.sum(x[a] * x[b], axis=-1, keepdims=True)
            sim = d / jnp.maximum(norms[a] * norms[b], eps)
            s = jnp.abs(sim)
            acc = acc + jnp.sum(jnp.where(s > threshold, s, 0.0))
    out_ref[...] = jnp.full(out_ref.shape, acc, jnp.float32)


def kernel(aspect, center, pairs, negs, offsets, lists):
    N, D = center.shape
    A = aspect.shape[0] // N
    B = pairs.shape[0]
    NN = negs.shape[1]
    L = lists.shape[0]
    bag = L // B
    threshold, reg_coef, eps = 0.3, 0.01, 1e-8

    centers = pairs[:, 0]
    contexts = pairs[:, 1]
    aoff = (jnp.arange(A, dtype=jnp.int32) * N)

    ctr_emb = center[centers]
    ctx_emb = aspect[contexts[:, None] + aoff].reshape(B, A * D)
    neg_emb = aspect[negs[:, :, None] + aoff].reshape(B, NN * A * D)
    bag_idx = lists.reshape(B, bag).T
    bag_emb = aspect[bag_idx[:, :, None] + aoff].reshape(bag, B, A * D)

    sum_a = jnp.asarray(_chunk_sum_mat(D, A))

    TB = 256 if B % 256 == 0 else B
    G = B // TB
    loss_fn = functools.partial(_loss_kernel, num_aspects=A, dim=D,
                                num_negs=NN, bag=bag,
                                inv_total=1.0 / float(B * A))
    sg_partials = pl.pallas_call(
        loss_fn,
        out_shape=jax.ShapeDtypeStruct((G, 8, 128), jnp.float32),
        grid=(G,),
        in_specs=[
            pl.BlockSpec((TB, D), lambda i: (i, 0)),
            pl.BlockSpec((bag, TB, A * D), lambda i: (0, i, 0)),
            pl.BlockSpec((TB, A * D), lambda i: (i, 0)),
            pl.BlockSpec((TB, NN * A * D), lambda i: (i, 0)),
            pl.BlockSpec((A * D, A), lambda i: (0, 0)),
        ],
        out_specs=pl.BlockSpec((1, 8, 128), lambda i: (i, 0, 0)),
        compiler_params=pltpu.CompilerParams(
            dimension_semantics=("parallel",),
            vmem_limit_bytes=48 * 1024 * 1024),
    )(ctr_emb, bag_emb, ctx_emb, neg_emb, sum_a)
    sg_loss = jnp.sum(sg_partials[:, 0, 0])

    TN = 4096 if N % 4096 == 0 else N
    GN = N // TN
    reg_fn = functools.partial(_reg_kernel, num_aspects=A,
                               threshold=threshold, eps=eps)
    div_partials = pl.pallas_call(
        reg_fn,
        out_shape=jax.ShapeDtypeStruct((GN, 8, 128), jnp.float32),
        grid=(GN,),
        in_specs=[pl.BlockSpec((A, TN, D), lambda i: (0, i, 0))],
        out_specs=pl.BlockSpec((1, 8, 128), lambda i: (i, 0, 0)),
        compiler_params=pltpu.CompilerParams(
            dimension_semantics=("parallel",),
            vmem_limit_bytes=48 * 1024 * 1024),
    )(aspect.reshape(A, N, D))
    div_metric = jnp.sum(div_partials[:, 0, 0])

    div_reg = reg_coef * div_metric
    return sg_loss + div_reg, div_reg
